# baseline (speedup 1.0000x reference)
import functools

import jax
import jax.numpy as jnp
from jax import lax
from jax.experimental import pallas as pl
from jax.experimental.pallas import tpu as pltpu
from jax.experimental.pallas import tpu_sc as plsc

NC = 2
NS = 16
NW = NC * NS
CHUNK = 128
G = 16
KSUB = 4


@functools.partial(jax.jit, static_argnums=(2,))
def _sc_gather(view16, idx4, R):
    nchunk = idx4.shape[1]
    rows_w = nchunk * CHUNK * KSUB
    mesh = plsc.VectorSubcoreMesh(
        core_axis_name="c", subcore_axis_name="s", num_cores=NC, num_subcores=NS
    )

    @functools.partial(
        pl.kernel,
        mesh=mesh,
        compiler_params=pltpu.CompilerParams(
            use_tc_tiling_on_sc=False, needs_layout_passes=False
        ),
        out_type=jax.ShapeDtypeStruct((R * KSUB, G), jnp.float32),
        scratch_types=[
            pltpu.VMEM((nchunk, KSUB, CHUNK), jnp.int32),
            pltpu.VMEM((KSUB * CHUNK, G), jnp.float32),
            pltpu.VMEM((KSUB * CHUNK, G), jnp.float32),
            pltpu.SemaphoreType.DMA,
            pltpu.SemaphoreType.DMA,
        ],
    )
    def k(view_hbm, idx4_hbm, out_hbm, idx4_v, b4a, b4b, sema, semb):
        wid = lax.axis_index("s") * NC + lax.axis_index("c")
        base = wid * rows_w
        pltpu.sync_copy(idx4_hbm.at[wid], idx4_v)

        def start4(j, buf, sem):
            for t in range(KSUB):
                pltpu.make_async_copy(
                    view_hbm.at[idx4_v.at[j, t]],
                    buf.at[pl.ds(t * CHUNK, CHUNK)],
                    sem,
                ).start()

        def wait4(j, buf, sem):
            for t in range(KSUB):
                pltpu.make_async_copy(
                    view_hbm.at[idx4_v.at[j, t]],
                    buf.at[pl.ds(t * CHUNK, CHUNK)],
                    sem,
                ).wait()

        start4(0, b4a, sema)

        def chunkbody(i, _):
            j0 = i * 2
            for jj, bb, ss, nb, ns in (
                (j0, b4a, sema, b4b, semb),
                (j0 + 1, b4b, semb, b4a, sema),
            ):
                @pl.when(jj + 1 < nchunk)
                def _():
                    start4(jj + 1, nb, ns)

                wait4(jj, bb, ss)
                pltpu.sync_copy(
                    bb, out_hbm.at[pl.ds(base + jj * CHUNK * KSUB, CHUNK * KSUB)]
                )
            return 0

        lax.fori_loop(0, nchunk // 2, chunkbody, 0)

    return k(view16, idx4)


def _mlp(emb_cat, num_inputs, W1a, W1b, b1, W2, b2, Wout, bout):
    B, E = emb_cat.shape
    NUMD = num_inputs.shape[1]
    H1 = W1a.shape[1]
    H2 = W2.shape[1]
    BLK = 1024

    def body(emb_ref, num_ref, w1a_ref, w1b_ref, b1_ref, w2_ref, b2_ref,
             wout_ref, bout_ref, out_ref):
        x1 = jnp.dot(emb_ref[...], w1a_ref[...], preferred_element_type=jnp.float32)
        x1 = x1 + jnp.dot(num_ref[...], w1b_ref[...], preferred_element_type=jnp.float32)
        h1 = jnp.maximum(x1 + b1_ref[...], 0.0)
        h2 = jnp.maximum(
            jnp.dot(h1, w2_ref[...], preferred_element_type=jnp.float32) + b2_ref[...],
            0.0,
        )
        out_ref[...] = (
            jnp.dot(h2, wout_ref[...], preferred_element_type=jnp.float32)
            + bout_ref[...]
        )

    out = pl.pallas_call(
        body,
        grid=(B // BLK,),
        in_specs=[
            pl.BlockSpec((BLK, E), lambda i: (i, 0)),
            pl.BlockSpec((BLK, NUMD), lambda i: (i, 0)),
            pl.BlockSpec((E, H1), lambda i: (0, 0)),
            pl.BlockSpec((NUMD, H1), lambda i: (0, 0)),
            pl.BlockSpec((1, H1), lambda i: (0, 0)),
            pl.BlockSpec((H1, H2), lambda i: (0, 0)),
            pl.BlockSpec((1, H2), lambda i: (0, 0)),
            pl.BlockSpec((H2, 1), lambda i: (0, 0)),
            pl.BlockSpec((1, 1), lambda i: (0, 0)),
        ],
        out_specs=pl.BlockSpec((BLK, 1), lambda i: (i, 0)),
        out_shape=jax.ShapeDtypeStruct((B, 1), jnp.float32),
    )(
        emb_cat,
        num_inputs,
        W1a,
        W1b,
        b1.reshape(1, H1),
        W2,
        b2.reshape(1, H2),
        Wout,
        bout.reshape(1, 1),
    )
    return out[:, 0]


def _repack(tt, V, D):
    F = tt.shape[0]
    VB = 512
    nvb = -(-V // VB)

    def body(in_ref, out_ref):
        x = in_ref[0]
        xt = jnp.swapaxes(x, 0, 1)
        out_ref[0] = jnp.pad(xt, ((0, 0), (0, 128 - D)))

    out = pl.pallas_call(
        body,
        grid=(F, nvb),
        in_specs=[pl.BlockSpec((1, D, VB), lambda f, v: (f, 0, v))],
        out_specs=pl.BlockSpec((1, VB, 128), lambda f, v: (f, v, 0)),
        out_shape=jax.ShapeDtypeStruct((F, V, 128), jnp.float32),
    )(tt)
    return out.reshape(F * V, 128)


def kernel(cat_inputs, num_inputs, tables, W1, b1, W2, b2, Wout, bout):
    F, V, D = tables.shape
    B = cat_inputs.shape[0]
    R = B * F
    DP = 64
    nchunk = R // (NW * CHUNK)
    tt = jnp.transpose(tables, (0, 2, 1))
    padded = _repack(tt, V, D)
    view16 = padded.reshape(F * V * 128 // G, G)
    idx = (cat_inputs.astype(jnp.int32)
           + jnp.arange(F, dtype=jnp.int32)[None, :] * V).reshape(-1)
    idx4 = ((idx * (128 // G))[:, None]
            + jnp.arange(KSUB, dtype=jnp.int32)[None, :])
    idx4 = idx4.reshape(NW, nchunk, KSUB, CHUNK)
    emb = _sc_gather(view16, idx4, R)
    emb_cat = emb.reshape(B, F * DP)
    W1a = W1[: F * D].reshape(F, D, -1)
    W1a = jnp.pad(W1a, ((0, 0), (0, DP - D), (0, 0))).reshape(F * DP, -1)
    W1b = W1[F * D :]
    return _mlp(emb_cat, num_inputs, W1a, W1b, b1, W2, b2, Wout, bout)


# speedup vs baseline: 1.7473x; 1.7473x over previous
import functools

import jax
import jax.numpy as jnp
from jax import lax
from jax.experimental import pallas as pl
from jax.experimental.pallas import tpu as pltpu
from jax.experimental.pallas import tpu_sc as plsc

NC = 2
NS = 16
NW = NC * NS
CHUNK = 128
G = 16
KSUB = 4


@functools.partial(jax.jit, static_argnums=(2,))
def _sc_gather(view16, idx4, R):
    nchunk = idx4.shape[1]
    rows_w = nchunk * CHUNK * KSUB
    mesh = plsc.VectorSubcoreMesh(
        core_axis_name="c", subcore_axis_name="s", num_cores=NC, num_subcores=NS
    )

    @functools.partial(
        pl.kernel,
        mesh=mesh,
        compiler_params=pltpu.CompilerParams(
            use_tc_tiling_on_sc=False, needs_layout_passes=False
        ),
        out_type=jax.ShapeDtypeStruct((R * KSUB, G), jnp.float32),
        scratch_types=[
            pltpu.VMEM((nchunk, KSUB, CHUNK), jnp.int32),
            pltpu.VMEM((KSUB * CHUNK, G), jnp.float32),
            pltpu.VMEM((KSUB * CHUNK, G), jnp.float32),
            pltpu.SemaphoreType.DMA,
            pltpu.SemaphoreType.DMA,
        ],
    )
    def k(view_hbm, idx4_hbm, out_hbm, idx4_v, b4a, b4b, sema, semb):
        wid = lax.axis_index("s") * NC + lax.axis_index("c")
        base = wid * rows_w
        pltpu.sync_copy(idx4_hbm.at[wid], idx4_v)

        def start4(j, buf, sem):
            for t in range(KSUB):
                pltpu.make_async_copy(
                    view_hbm.at[idx4_v.at[j, t]],
                    buf.at[pl.ds(t * CHUNK, CHUNK)],
                    sem,
                ).start()

        def wait4(j, buf, sem):
            for t in range(KSUB):
                pltpu.make_async_copy(
                    view_hbm.at[idx4_v.at[j, t]],
                    buf.at[pl.ds(t * CHUNK, CHUNK)],
                    sem,
                ).wait()

        start4(0, b4a, sema)

        def chunkbody(i, _):
            j0 = i * 2
            for jj, bb, ss, nb, ns in (
                (j0, b4a, sema, b4b, semb),
                (j0 + 1, b4b, semb, b4a, sema),
            ):
                @pl.when(jj + 1 < nchunk)
                def _():
                    start4(jj + 1, nb, ns)

                wait4(jj, bb, ss)
                pltpu.sync_copy(
                    bb, out_hbm.at[pl.ds(base + jj * CHUNK * KSUB, CHUNK * KSUB)]
                )
            return 0

        lax.fori_loop(0, nchunk // 2, chunkbody, 0)

    return k(view16, idx4)


def _mlp(emb_cat, num_inputs, W1a, W1b, b1, W2, b2, Wout, bout):
    B, E = emb_cat.shape
    NUMD = num_inputs.shape[1]
    H1 = W1a.shape[1]
    H2 = W2.shape[1]
    BLK = 1024

    def body(emb_ref, num_ref, w1a_ref, w1b_ref, b1_ref, w2_ref, b2_ref,
             wout_ref, bout_ref, out_ref):
        x1 = jnp.dot(emb_ref[...], w1a_ref[...], preferred_element_type=jnp.float32)
        x1 = x1 + jnp.dot(num_ref[...], w1b_ref[...], preferred_element_type=jnp.float32)
        h1 = jnp.maximum(x1 + b1_ref[...], 0.0)
        h2 = jnp.maximum(
            jnp.dot(h1, w2_ref[...], preferred_element_type=jnp.float32) + b2_ref[...],
            0.0,
        )
        out_ref[...] = (
            jnp.dot(h2, wout_ref[...], preferred_element_type=jnp.float32)
            + bout_ref[...]
        )

    out = pl.pallas_call(
        body,
        grid=(B // BLK,),
        in_specs=[
            pl.BlockSpec((BLK, E), lambda i: (i, 0)),
            pl.BlockSpec((BLK, NUMD), lambda i: (i, 0)),
            pl.BlockSpec((E, H1), lambda i: (0, 0)),
            pl.BlockSpec((NUMD, H1), lambda i: (0, 0)),
            pl.BlockSpec((1, H1), lambda i: (0, 0)),
            pl.BlockSpec((H1, H2), lambda i: (0, 0)),
            pl.BlockSpec((1, H2), lambda i: (0, 0)),
            pl.BlockSpec((H2, 1), lambda i: (0, 0)),
            pl.BlockSpec((1, 1), lambda i: (0, 0)),
        ],
        out_specs=pl.BlockSpec((BLK, 1), lambda i: (i, 0)),
        out_shape=jax.ShapeDtypeStruct((B, 1), jnp.float32),
    )(
        emb_cat,
        num_inputs,
        W1a,
        W1b,
        b1.reshape(1, H1),
        W2,
        b2.reshape(1, H2),
        Wout,
        bout.reshape(1, 1),
    )
    return out[:, 0]


def kernel(cat_inputs, num_inputs, tables, W1, b1, W2, b2, Wout, bout):
    F, V, D = tables.shape
    B = cat_inputs.shape[0]
    R = B * F
    DP = 64
    nchunk = R // (NW * CHUNK)
    padded = jnp.pad(tables.reshape(F * V, D), ((0, 0), (0, 128 - D)))
    view16 = padded.reshape(F * V * 128 // G, G)
    idx = (cat_inputs.astype(jnp.int32)
           + jnp.arange(F, dtype=jnp.int32)[None, :] * V).reshape(-1)
    idx4 = ((idx * (128 // G))[:, None]
            + jnp.arange(KSUB, dtype=jnp.int32)[None, :])
    idx4 = idx4.reshape(NW, nchunk, KSUB, CHUNK)
    emb = _sc_gather(view16, idx4, R)
    emb_cat = emb.reshape(B, F * DP)
    W1a = W1[: F * D].reshape(F, D, -1)
    W1a = jnp.pad(W1a, ((0, 0), (0, DP - D), (0, 0))).reshape(F * DP, -1)
    W1b = W1[F * D :]
    return _mlp(emb_cat, num_inputs, W1a, W1b, b1, W2, b2, Wout, bout)
